# baseline (speedup 1.0000x reference)
import jax
import jax.numpy as jnp
from jax import lax
from jax.experimental import pallas as pl
from jax.experimental.pallas import tpu as pltpu
from jax.experimental.pallas import tpu_sc as plsc

N_NODES = 10000
N_EDGES = 320000
D_FEAT = 128

_NC = 2
_NS = 16
_NW = _NC * _NS
_EPW = N_EDGES // _NW
_CHUNK = 80
_NCHUNK = _EPW // _CHUNK
_G = _CHUNK // 16
_NBUF = 2


def _normalize_body(x_ref, o_ref):
    x = x_ref[...]
    n = jnp.sqrt(jnp.sum(x * x, axis=1, keepdims=True))
    o_ref[...] = (x / jnp.maximum(n, 1e-12)).astype(jnp.bfloat16)


def _normalize(x):
    blk = 1000
    return pl.pallas_call(
        _normalize_body,
        out_shape=jax.ShapeDtypeStruct(x.shape, jnp.bfloat16),
        grid=(x.shape[0] // blk,),
        in_specs=[pl.BlockSpec((blk, x.shape[1]), lambda i: (i, 0))],
        out_specs=pl.BlockSpec((blk, x.shape[1]), lambda i: (i, 0)),
    )(x)


def _sc_body(nu_hbm, nv_hbm, src_hbm, dst_hbm, out_hbm,
             sidx, didx, nu_sp, nv_sp, urows, vrows, obuf, psum, sems, isems):
    cid = lax.axis_index("c")
    sid = lax.axis_index("s")
    wid = sid * _NC + cid
    base = wid * _EPW
    row_iota = lax.iota(jnp.int32, 16)

    rows_per_tile = N_NODES // _NS
    stripe = pl.ds(sid * rows_per_tile, rows_per_tile)
    pltpu.sync_copy(nu_hbm.at[stripe], nu_sp.at[stripe])
    pltpu.sync_copy(nv_hbm.at[stripe], nv_sp.at[stripe])
    plsc.subcore_barrier()

    def idx_start(ci, p):
        h = pl.ds(base + ci * _CHUNK, _CHUNK)
        pltpu.async_copy(src_hbm.at[h], sidx.at[p], isems.at[p])
        pltpu.async_copy(dst_hbm.at[h], didx.at[p], isems.at[p])

    def idx_wait(ci, p):
        h = pl.ds(base + ci * _CHUNK, _CHUNK)
        pltpu.make_async_copy(src_hbm.at[h], sidx.at[p], isems.at[p]).wait()
        pltpu.make_async_copy(dst_hbm.at[h], didx.at[p], isems.at[p]).wait()

    def start(ci, buf):
        pltpu.async_copy(nu_sp.at[sidx.at[buf]], urows.at[buf], sems.at[buf])
        pltpu.async_copy(nv_sp.at[didx.at[buf]], vrows.at[buf], sems.at[buf])

    def wait(ci, buf):
        pltpu.make_async_copy(
            nu_sp.at[sidx.at[buf]], urows.at[buf], sems.at[buf]).wait()
        pltpu.make_async_copy(
            nv_sp.at[didx.at[buf]], vrows.at[buf], sems.at[buf]).wait()

    def _tree_sum(vals):
        while len(vals) > 1:
            vals = [a + b for a, b in zip(vals[::2], vals[1::2])]
        return vals[0]

    def compute(ci, buf):
        def group_body(g, carry):
            accs = []
            for k in range(16):
                e = g * 16 + k
                prods = []
                for j in range(4):
                    uw = urows[buf, e, pl.ds(j * 32, 32)]
                    vw = vrows[buf, e, pl.ds(j * 32, 32)]
                    prods.append(uw * vw)
                acc32 = _tree_sum(prods)
                pa, pb = plsc.unpack(acc32, format=plsc.PackFormat.INTERLEAVED)
                accs.append(pa + pb)
            for k in range(16):
                psum[pl.ds(k * 16, 16)] = accs[k]
            cols = [plsc.load_gather(psum, [row_iota * 16 + j])
                    for j in range(16)]
            obuf[pl.ds(ci * _CHUNK + g * 16, 16)] = _tree_sum(cols)
            return carry

        lax.fori_loop(0, _G, group_body, 0)

    idx_start(0, 0)
    idx_wait(0, 0)
    start(0, 0)
    idx_start(1, 1)

    def body2(i, carry):
        c0 = 2 * i
        c1 = c0 + 1
        idx_wait(c1, 1)
        wait(c0, 0)
        start(c1, 1)
        idx_start(c0 + 2, 0)
        compute(c0, 0)
        idx_wait(c0 + 2, 0)
        wait(c1, 1)
        start(c0 + 2, 0)

        @pl.when(c0 + 3 < _NCHUNK)
        def _():
            idx_start(c0 + 3, 1)

        compute(c1, 1)
        return carry

    lax.fori_loop(0, (_NCHUNK - 1) // 2, body2, 0)
    wait(_NCHUNK - 1, 0)
    compute(_NCHUNK - 1, 0)

    pltpu.sync_copy(obuf, out_hbm.at[pl.ds(base, _EPW)])


def _sc_cosine(nu, nv, src, dst):
    mesh = plsc.VectorSubcoreMesh(core_axis_name="c", subcore_axis_name="s")
    f = pl.kernel(
        _sc_body,
        mesh=mesh,
        compiler_params=pltpu.CompilerParams(
            needs_layout_passes=False,
            use_tc_tiling_on_sc=False,
        ),
        out_type=jax.ShapeDtypeStruct((N_EDGES,), jnp.float32),
        scratch_types=[
            pltpu.VMEM((_NBUF, _CHUNK), jnp.int32),
            pltpu.VMEM((_NBUF, _CHUNK), jnp.int32),
            pltpu.VMEM_SHARED((N_NODES, D_FEAT), jnp.bfloat16),
            pltpu.VMEM_SHARED((N_NODES, D_FEAT), jnp.bfloat16),
            pltpu.VMEM((_NBUF, _CHUNK, D_FEAT), jnp.bfloat16),
            pltpu.VMEM((_NBUF, _CHUNK, D_FEAT), jnp.bfloat16),
            pltpu.VMEM((_EPW,), jnp.float32),
            pltpu.VMEM((256,), jnp.float32),
            pltpu.SemaphoreType.DMA((_NBUF,)),
            pltpu.SemaphoreType.DMA((_NBUF,)),
        ],
    )
    return f(nu, nv, src, dst)


def kernel(h_user, h_item, edge_index):
    nu = _normalize(h_user)
    nv = _normalize(h_item)
    src = edge_index[0]
    dst = edge_index[1]
    cos = _sc_cosine(nu, nv, src, dst)
    return cos.reshape(N_EDGES, 1)


# speedup vs baseline: 1.0227x; 1.0227x over previous
import jax
import jax.numpy as jnp
from jax import lax
from jax.experimental import pallas as pl
from jax.experimental.pallas import tpu as pltpu
from jax.experimental.pallas import tpu_sc as plsc

N_NODES = 10000
N_EDGES = 320000
D_FEAT = 128

_NC = 2
_NS = 16
_NW = _NC * _NS
_EPW = N_EDGES // _NW
_CHUNK = 80
_NCHUNK = _EPW // _CHUNK
_G = _CHUNK // 16
_NBUF = 5


def _normalize_body(x_ref, o_ref):
    x = x_ref[...]
    n = jnp.sqrt(jnp.sum(x * x, axis=1, keepdims=True))
    o_ref[...] = (x / jnp.maximum(n, 1e-12)).astype(jnp.bfloat16)


def _normalize(x):
    blk = 1000
    return pl.pallas_call(
        _normalize_body,
        out_shape=jax.ShapeDtypeStruct(x.shape, jnp.bfloat16),
        grid=(x.shape[0] // blk,),
        in_specs=[pl.BlockSpec((blk, x.shape[1]), lambda i: (i, 0))],
        out_specs=pl.BlockSpec((blk, x.shape[1]), lambda i: (i, 0)),
    )(x)


def _sc_body(nu_hbm, nv_hbm, src_hbm, dst_hbm, out_hbm,
             sidx, didx, urows, vrows, obuf, psum, sems):
    cid = lax.axis_index("c")
    sid = lax.axis_index("s")
    wid = sid * _NC + cid
    base = wid * _EPW
    row_iota = lax.iota(jnp.int32, 16)

    pltpu.sync_copy(src_hbm.at[pl.ds(base, _EPW)], sidx)
    pltpu.sync_copy(dst_hbm.at[pl.ds(base, _EPW)], didx)

    def start(ci, buf):
        idx = pl.ds(ci * _CHUNK, _CHUNK)
        pltpu.async_copy(nu_hbm.at[sidx.at[idx]], urows.at[buf], sems.at[buf])
        pltpu.async_copy(nv_hbm.at[didx.at[idx]], vrows.at[buf], sems.at[buf])

    def wait(ci, buf):
        idx = pl.ds(ci * _CHUNK, _CHUNK)
        pltpu.make_async_copy(
            nu_hbm.at[sidx.at[idx]], urows.at[buf], sems.at[buf]).wait()
        pltpu.make_async_copy(
            nv_hbm.at[didx.at[idx]], vrows.at[buf], sems.at[buf]).wait()

    def _tree_sum(vals):
        while len(vals) > 1:
            vals = [a + b for a, b in zip(vals[::2], vals[1::2])]
        return vals[0]

    def compute(ci, buf):
        def group_body(g, carry):
            accs = []
            for k in range(16):
                e = g * 16 + k
                prods = []
                for j in range(4):
                    uw = urows[buf, e, pl.ds(j * 32, 32)]
                    vw = vrows[buf, e, pl.ds(j * 32, 32)]
                    prods.append(uw * vw)
                acc32 = _tree_sum(prods)
                pa, pb = plsc.unpack(acc32, format=plsc.PackFormat.INTERLEAVED)
                accs.append(pa + pb)
            for k in range(16):
                psum[pl.ds(k * 16, 16)] = accs[k]
            cols = [plsc.load_gather(psum, [row_iota * 16 + j])
                    for j in range(16)]
            obuf[pl.ds(ci * _CHUNK + g * 16, 16)] = _tree_sum(cols)
            return carry

        lax.fori_loop(0, _G, group_body, 0)

    def guarded_start(ci, buf):
        @pl.when(ci < _NCHUNK)
        def _():
            start(ci, buf)

    start(0, 0)
    start(1, 1)
    start(2, 2)
    start(3, 3)

    def body5(i, carry):
        c0 = 5 * i
        start(c0 + 4, 4)
        wait(c0, 0)
        compute(c0, 0)
        guarded_start(c0 + 5, 0)
        wait(c0 + 1, 1)
        compute(c0 + 1, 1)
        guarded_start(c0 + 6, 1)
        wait(c0 + 2, 2)
        compute(c0 + 2, 2)
        guarded_start(c0 + 7, 2)
        wait(c0 + 3, 3)
        compute(c0 + 3, 3)
        guarded_start(c0 + 8, 3)
        wait(c0 + 4, 4)
        compute(c0 + 4, 4)
        return carry

    lax.fori_loop(0, _NCHUNK // _NBUF, body5, 0)

    pltpu.sync_copy(obuf, out_hbm.at[pl.ds(base, _EPW)])


def _sc_cosine(nu, nv, src, dst):
    mesh = plsc.VectorSubcoreMesh(core_axis_name="c", subcore_axis_name="s")
    f = pl.kernel(
        _sc_body,
        mesh=mesh,
        compiler_params=pltpu.CompilerParams(
            needs_layout_passes=False,
            use_tc_tiling_on_sc=False,
        ),
        out_type=jax.ShapeDtypeStruct((N_EDGES,), jnp.float32),
        scratch_types=[
            pltpu.VMEM((_EPW,), jnp.int32),
            pltpu.VMEM((_EPW,), jnp.int32),
            pltpu.VMEM((_NBUF, _CHUNK, D_FEAT), jnp.bfloat16),
            pltpu.VMEM((_NBUF, _CHUNK, D_FEAT), jnp.bfloat16),
            pltpu.VMEM((_EPW,), jnp.float32),
            pltpu.VMEM((256,), jnp.float32),
            pltpu.SemaphoreType.DMA((_NBUF,)),
        ],
    )
    return f(nu, nv, src, dst)


def kernel(h_user, h_item, edge_index):
    nu = _normalize(h_user)
    nv = _normalize(h_item)
    src = edge_index[0]
    dst = edge_index[1]
    cos = _sc_cosine(nu, nv, src, dst)
    return cos.reshape(N_EDGES, 1)
